# baseline (speedup 1.0000x reference)
import functools

import jax
import jax.numpy as jnp
from jax import lax
from jax.experimental import pallas as pl
from jax.experimental.pallas import tpu as pltpu
from jax.experimental.pallas import tpu_sc as plsc

N_ROWS = 2_000_000
EMB = 32
N_TBL = 21 * 4 * 7
NW = 32
QUOTA = 62464
C = 256
CHUNKS = QUOTA // C
PAIRS = CHUNKS // 2
REM = N_ROWS - NW * QUOTA
REM_C = 128


def _sc_body(feat_hbm, tbl_hbm, wc_hbm, out_hbm,
             f0, f1, o0, o1, tbl_v, wc_v,
             si0, si1, so0, so1):
    wid = lax.axis_index("s") * 2 + lax.axis_index("c")
    pltpu.sync_copy(tbl_hbm, tbl_v)
    pltpu.sync_copy(wc_hbm, wc_v)
    row0 = wid * QUOTA
    fbuf = (f0, f1)
    obuf = (o0, o1)
    isem = (si0, si1)
    osem = (so0, so1)

    def in_copy(ci, b):
        return pltpu.make_async_copy(
            feat_hbm.at[:, pl.ds(row0 + ci * C, C)], fbuf[b], isem[b])

    def out_copy(ci, b):
        return pltpu.make_async_copy(
            obuf[b], out_hbm.at[:, pl.ds(row0 + ci * C, C)], osem[b])

    def compute(feats_v, out_v, ngroups):
        @plsc.parallel_loop(0, ngroups, 1, unroll=4)
        def group(g):
            s = g * 16
            af = feats_v[0, pl.ds(s, 16)]
            of = feats_v[1, pl.ds(s, 16)]
            mf = feats_v[2, pl.ds(s, 16)]
            tv = (af.astype(jnp.int32) * 28 + of.astype(jnp.int32) * 7
                  + mf.astype(jnp.int32) + 3)
            tv = jnp.clip(tv, 0, N_TBL - 1)
            tw = tv * (EMB + 1)
            ck = [feats_v[3 + k, pl.ds(s, 16)] for k in range(5)]
            for j in range(EMB):
                acc = plsc.load_gather(tbl_v, [tw + j])
                for k in range(5):
                    acc = acc + wc_v[j * 5 + k, pl.ds(0, 16)] * ck[k]
                out_v[j, pl.ds(s, 16)] = acc

    in_copy(0, 0).start()
    in_copy(1, 1).start()

    def pair(i, carry):
        for b in range(2):
            ci = 2 * i + b
            in_copy(ci, b).wait()

            @pl.when(i >= 1)
            def _():
                out_copy(ci - 2, b).wait()

            compute(fbuf[b], obuf[b], C // 16)

            @pl.when(ci + 2 < CHUNKS)
            def _():
                in_copy(ci + 2, b).start()

            out_copy(ci, b).start()
        return carry

    lax.fori_loop(0, PAIRS, pair, 0)
    out_copy(CHUNKS - 2, 0).wait()
    out_copy(CHUNKS - 1, 1).wait()

    @pl.when(wid == NW - 1)
    def _():
        def rem_chunk(ri, carry):
            rbase = NW * QUOTA + ri * REM_C
            pltpu.sync_copy(feat_hbm.at[:, pl.ds(rbase, REM_C)],
                            f0.at[:, pl.ds(0, REM_C)])
            compute(f0, o0, REM_C // 16)
            pltpu.sync_copy(o0.at[:, pl.ds(0, REM_C)],
                            out_hbm.at[:, pl.ds(rbase, REM_C)])
            return carry

        lax.fori_loop(0, REM // REM_C, rem_chunk, 0)


@jax.jit
def _sc_call(feats_t, tbl, wc):
    mesh = plsc.VectorSubcoreMesh(core_axis_name="c", subcore_axis_name="s")
    f = pl.kernel(
        _sc_body,
        mesh=mesh,
        compiler_params=pltpu.CompilerParams(needs_layout_passes=False,
                                             use_tc_tiling_on_sc=True),
        out_type=jax.ShapeDtypeStruct((EMB, N_ROWS), jnp.float32),
        scratch_types=[
            pltpu.VMEM((8, C), jnp.float32),
            pltpu.VMEM((8, C), jnp.float32),
            pltpu.VMEM((EMB, C), jnp.float32),
            pltpu.VMEM((EMB, C), jnp.float32),
            pltpu.VMEM((N_TBL * (EMB + 1),), jnp.float32),
            pltpu.VMEM((EMB * 5, 16), jnp.float32),
            pltpu.SemaphoreType.DMA,
            pltpu.SemaphoreType.DMA,
            pltpu.SemaphoreType.DMA,
            pltpu.SemaphoreType.DMA,
        ],
    )
    return f(feats_t, tbl, wc)


def kernel(orbital_features, atomic_table, orbital_table, m_table, W, b):
    A2 = atomic_table @ W[:, 5:37].T
    O2 = orbital_table @ W[:, 37:53].T
    M2 = m_table @ W[:, 53:61].T
    T = (A2[:, None, None, :] + O2[None, :, None, :] + M2[None, None, :, :]
         + b).reshape(N_TBL, EMB).astype(jnp.float32)
    T = jnp.pad(T, ((0, 0), (0, 1))).reshape(N_TBL * (EMB + 1))
    Wc = jnp.broadcast_to(W[:, :5].astype(jnp.float32).reshape(EMB * 5, 1),
                          (EMB * 5, 16))
    feats_t = orbital_features.T.astype(jnp.float32)
    return _sc_call(feats_t, T, Wc).T


# speedup vs baseline: 1.4943x; 1.4943x over previous
import functools

import jax
import jax.numpy as jnp
from jax import lax
from jax.experimental import pallas as pl
from jax.experimental.pallas import tpu as pltpu
from jax.experimental.pallas import tpu_sc as plsc

N_ROWS = 2_000_000
EMB = 32
N_TBL = 21 * 4 * 7
NW = 32
QUOTA = 62464
C = 256
CHUNKS = QUOTA // C
PAIRS = CHUNKS // 2
REM = N_ROWS - NW * QUOTA
REM_C = 128


def _sc_body(feat_hbm, tbl_hbm, wc_hbm, out_hbm,
             f0, f1, o0, o1, tbl_v, wc_v,
             si0, si1, so0, so1):
    wid = lax.axis_index("s") * 2 + lax.axis_index("c")
    pltpu.sync_copy(tbl_hbm, tbl_v)
    pltpu.sync_copy(wc_hbm, wc_v)
    w_lo = [wc_v[k, pl.ds(0, 16)] for k in range(5)]
    w_hi = [wc_v[k, pl.ds(16, 16)] for k in range(5)]
    lanes = lax.iota(jnp.int32, 16)
    lanes16 = lanes + 16
    zeros = jnp.zeros((16,), jnp.int32)
    row0 = wid * QUOTA
    fbuf = (f0, f1)
    obuf = (o0, o1)
    isem = (si0, si1)
    osem = (so0, so1)

    def in_copy(ci, b):
        return pltpu.make_async_copy(
            feat_hbm.at[:, pl.ds(row0 + ci * C, C)], fbuf[b], isem[b])

    def out_copy(ci, b):
        return pltpu.make_async_copy(
            obuf[b], out_hbm.at[:, pl.ds(row0 + ci * C, C)], osem[b])

    def compute(feats_v, out_v, ngroups):
        @plsc.parallel_loop(0, ngroups, 1, unroll=4)
        def group(g):
            s = g * 16
            af = feats_v[0, pl.ds(s, 16)]
            of = feats_v[1, pl.ds(s, 16)]
            mf = feats_v[2, pl.ds(s, 16)]
            tv = (af.astype(jnp.int32) * 28 + of.astype(jnp.int32) * 7
                  + mf.astype(jnp.int32) + 3)
            tv = jnp.clip(tv, 0, N_TBL - 1)
            tw = tv * (EMB + 1)
            ck = [feats_v[3 + k, pl.ds(s, 16)] for k in range(5)]
            for j in range(EMB):
                acc = plsc.load_gather(tbl_v, [tw + j])
                wh = w_lo if j < 16 else w_hi
                for k in range(5):
                    acc = acc + wh[k][j % 16] * ck[k]
                out_v[j, pl.ds(s, 16)] = acc

    in_copy(0, 0).start()
    in_copy(1, 1).start()

    def pair(i, carry):
        for b in range(2):
            ci = 2 * i + b
            in_copy(ci, b).wait()

            @pl.when(i >= 1)
            def _():
                out_copy(ci - 2, b).wait()

            compute(fbuf[b], obuf[b], C // 16)

            @pl.when(ci + 2 < CHUNKS)
            def _():
                in_copy(ci + 2, b).start()

            out_copy(ci, b).start()
        return carry

    lax.fori_loop(0, PAIRS, pair, 0)
    out_copy(CHUNKS - 2, 0).wait()
    out_copy(CHUNKS - 1, 1).wait()

    @pl.when(wid == NW - 1)
    def _():
        def rem_chunk(ri, carry):
            rbase = NW * QUOTA + ri * REM_C
            pltpu.sync_copy(feat_hbm.at[:, pl.ds(rbase, REM_C)],
                            f0.at[:, pl.ds(0, REM_C)])
            compute(f0, o0, REM_C // 16)
            pltpu.sync_copy(o0.at[:, pl.ds(0, REM_C)],
                            out_hbm.at[:, pl.ds(rbase, REM_C)])
            return carry

        lax.fori_loop(0, REM // REM_C, rem_chunk, 0)


@jax.jit
def _sc_call(feats_t, tbl, wc):
    mesh = plsc.VectorSubcoreMesh(core_axis_name="c", subcore_axis_name="s")
    f = pl.kernel(
        _sc_body,
        mesh=mesh,
        compiler_params=pltpu.CompilerParams(needs_layout_passes=False,
                                             use_tc_tiling_on_sc=True),
        out_type=jax.ShapeDtypeStruct((EMB, N_ROWS), jnp.float32),
        scratch_types=[
            pltpu.VMEM((8, C), jnp.float32),
            pltpu.VMEM((8, C), jnp.float32),
            pltpu.VMEM((EMB, C), jnp.float32),
            pltpu.VMEM((EMB, C), jnp.float32),
            pltpu.VMEM((N_TBL * (EMB + 1),), jnp.float32),
            pltpu.VMEM((5, EMB), jnp.float32),
            pltpu.SemaphoreType.DMA,
            pltpu.SemaphoreType.DMA,
            pltpu.SemaphoreType.DMA,
            pltpu.SemaphoreType.DMA,
        ],
    )
    return f(feats_t, tbl, wc)


def kernel(orbital_features, atomic_table, orbital_table, m_table, W, b):
    A2 = atomic_table @ W[:, 5:37].T
    O2 = orbital_table @ W[:, 37:53].T
    M2 = m_table @ W[:, 53:61].T
    T = (A2[:, None, None, :] + O2[None, :, None, :] + M2[None, None, :, :]
         + b).reshape(N_TBL, EMB).astype(jnp.float32)
    T = jnp.pad(T, ((0, 0), (0, 1))).reshape(N_TBL * (EMB + 1))
    Wc = W[:, :5].T.astype(jnp.float32)
    feats_t = orbital_features.T.astype(jnp.float32)
    return _sc_call(feats_t, T, Wc).T
